# baseline (speedup 1.0000x reference)
import functools

import jax
import jax.numpy as jnp
from jax import lax
from jax.experimental import pallas as pl
from jax.experimental.pallas import tpu as pltpu
from jax.experimental.pallas import tpu_sc as plsc

NS = 16
NC = 2


def _sc_degree(dst, n, h, ch):
    e = dst.shape[0]
    round_e = NC * NS * ch
    n_chunks = -(-e // round_e)
    n_chunks += n_chunks % 2
    n_pairs = n_chunks // 2
    epad = round_e * n_chunks
    pad = epad - e
    if pad:
        fill = jnp.arange(pad, dtype=jnp.int32)
        dst = jnp.concatenate([dst, n + (fill % 8)])
    npad = n + 8
    per_tile = epad // (NC * NS)
    rw = 80
    n_rw = n // rw
    mesh = plsc.VectorSubcoreMesh(core_axis_name="c", subcore_axis_name="s")

    @functools.partial(
        pl.kernel,
        out_type=jax.ShapeDtypeStruct((NC * n, h), jnp.float32),
        mesh=mesh,
        scratch_types=[
            pltpu.VMEM((ch,), jnp.int32),
            pltpu.VMEM((ch,), jnp.int32),
            pltpu.VMEM((ch, h), jnp.float32),
            pltpu.VMEM((rw, h), jnp.float32),
            pltpu.VMEM_SHARED((npad, h), jnp.float32),
            pltpu.SemaphoreType.DMA,
            pltpu.SemaphoreType.DMA,
        ],
    )
    def deg_k(dst_hbm, out, didx0, didx1, ones_v, stage, acc,
              sem_i0, sem_i1):
        c = lax.axis_index("c")
        s = lax.axis_index("s")

        def fill_ones(i, carry):
            for j in range(h // 16):
                ones_v[i, pl.ds(j * 16, 16)] = jnp.ones((16,), jnp.float32)
            return carry

        lax.fori_loop(0, ch, fill_ones, 0)

        def fill_zeros(i, carry):
            for j in range(h // 16):
                stage[i, pl.ds(j * 16, 16)] = jnp.zeros((16,), jnp.float32)
            return carry

        lax.fori_loop(0, rw, fill_zeros, 0)
        for kk in range((n_rw + NS - 1) // NS):
            k = s + NS * kk

            @pl.when(k < n_rw)
            def _zero():
                pltpu.sync_copy(stage, acc.at[pl.ds(k * rw, rw)])

        plsc.subcore_barrier()
        base = (c * NS + s) * per_tile

        def eoff(ck):
            return pl.ds(base + ck * ch, ch)

        pltpu.async_copy(dst_hbm.at[eoff(0)], didx0, sem_i0)

        def body(t, carry):
            pltpu.async_copy(dst_hbm.at[eoff(2 * t + 1)], didx1, sem_i1)
            pltpu.make_async_copy(dst_hbm.at[eoff(0)], didx0, sem_i0).wait()
            pltpu.sync_copy(ones_v, acc.at[didx0], add=True)

            @pl.when(t < n_pairs - 1)
            def _prefetch0():
                pltpu.async_copy(dst_hbm.at[eoff(2 * t + 2)], didx0, sem_i0)

            pltpu.make_async_copy(dst_hbm.at[eoff(0)], didx1, sem_i1).wait()
            pltpu.sync_copy(ones_v, acc.at[didx1], add=True)
            return carry

        lax.fori_loop(0, n_pairs, body, 0)
        plsc.subcore_barrier()
        for kk in range((n_rw + NS - 1) // NS):
            k = s + NS * kk

            @pl.when(k < n_rw)
            def _dump():
                pltpu.sync_copy(acc.at[pl.ds(k * rw, rw)], stage)
                off = pl.multiple_of(c * n + k * rw, 8)
                pltpu.sync_copy(stage, out.at[pl.ds(off, rw)])

    return deg_k(dst)


def _sc_aggregate(g2, src, dst, ch):
    _, n, hw = g2.shape
    g2f = g2.reshape(NC * n, hw)
    e = src.shape[0]
    round_e = NS * ch
    n_chunks = -(-e // round_e)
    n_chunks += n_chunks % 2
    n_pairs = n_chunks // 2
    epad = round_e * n_chunks
    pad = epad - e
    if pad:
        fill = jnp.arange(pad, dtype=jnp.int32)
        src = jnp.concatenate([src, fill % n])
        dst = jnp.concatenate([dst, n + (fill % 8)])
    src_both = jnp.concatenate([src, src + n])
    npad = n + 8
    per_tile = epad // NS
    rw = 80
    n_rw = n // rw
    mesh = plsc.VectorSubcoreMesh(core_axis_name="c", subcore_axis_name="s")

    @functools.partial(
        pl.kernel,
        out_type=jax.ShapeDtypeStruct((NC, n, hw), jnp.float32),
        mesh=mesh,
        compiler_params=pltpu.CompilerParams(use_tc_tiling_on_sc=False),
        scratch_types=[
            pltpu.VMEM((ch,), jnp.int32),
            pltpu.VMEM((ch,), jnp.int32),
            pltpu.VMEM((ch,), jnp.int32),
            pltpu.VMEM((ch,), jnp.int32),
            pltpu.VMEM((ch, hw), jnp.float32),
            pltpu.VMEM((ch, hw), jnp.float32),
            pltpu.VMEM_SHARED((npad, hw), jnp.float32),
            pltpu.SemaphoreType.DMA,
            pltpu.SemaphoreType.DMA,
            pltpu.SemaphoreType.DMA,
            pltpu.SemaphoreType.DMA,
            pltpu.SemaphoreType.DMA,
            pltpu.SemaphoreType.DMA,
        ],
    )
    def scat_k(g_hbm, src_hbm, dst_hbm, out,
               sidx0, didx0, sidx1, didx1, rows0, rows1, acc,
               sem_g0, sem_g1, sem_i0, sem_i1, sem_s0, sem_s1):
        c = lax.axis_index("c")
        s = lax.axis_index("s")

        def zbody(i, carry):
            for j in range(hw // 16):
                rows0[i, pl.ds(j * 16, 16)] = jnp.zeros((16,), jnp.float32)
            return carry

        lax.fori_loop(0, rw, zbody, 0)
        stage = rows0.at[pl.ds(0, rw)]
        for kk in range((n_rw + NS - 1) // NS):
            k = s + NS * kk

            @pl.when(k < n_rw)
            def _zero():
                pltpu.sync_copy(stage, acc.at[pl.ds(k * rw, rw)])

        plsc.subcore_barrier()
        dbase = s * per_tile
        sbase = c * epad + dbase

        def soff(ck):
            return pl.ds(sbase + ck * ch, ch)

        def doff(ck):
            return pl.ds(dbase + ck * ch, ch)

        def start_idx(ck, sidx, didx, sem):
            pltpu.async_copy(src_hbm.at[soff(ck)], sidx, sem)
            pltpu.async_copy(dst_hbm.at[doff(ck)], didx, sem)

        def wait_idx(ck, sidx, didx, sem):
            pltpu.make_async_copy(src_hbm.at[soff(ck)], sidx, sem).wait()
            pltpu.make_async_copy(dst_hbm.at[doff(ck)], didx, sem).wait()

        def start_gather(sidx, rows, sem):
            pltpu.async_copy(g_hbm.at[sidx], rows, sem)

        def wait_gather(sidx, rows, sem):
            pltpu.make_async_copy(g_hbm.at[sidx], rows, sem).wait()

        def start_scatter(rows, didx, sem):
            pltpu.async_copy(rows, acc.at[didx], sem, add=True)

        def wait_scatter(rows, didx, sem):
            pltpu.make_async_copy(rows, acc.at[didx], sem).wait()

        start_idx(0, sidx0, didx0, sem_i0)
        wait_idx(0, sidx0, didx0, sem_i0)
        start_gather(sidx0, rows0, sem_g0)

        def body(t, carry):
            @pl.when(t > 0)
            def _drain1():
                wait_scatter(rows1, didx1, sem_s1)

            start_idx(2 * t + 1, sidx1, didx1, sem_i1)
            wait_gather(sidx0, rows0, sem_g0)
            start_scatter(rows0, didx0, sem_s0)
            wait_idx(2 * t + 1, sidx1, didx1, sem_i1)
            start_gather(sidx1, rows1, sem_g1)
            wait_scatter(rows0, didx0, sem_s0)

            @pl.when(t < n_pairs - 1)
            def _prefetch0():
                start_idx(2 * t + 2, sidx0, didx0, sem_i0)

            wait_gather(sidx1, rows1, sem_g1)
            start_scatter(rows1, didx1, sem_s1)

            @pl.when(t < n_pairs - 1)
            def _gather0():
                wait_idx(2 * t + 2, sidx0, didx0, sem_i0)
                start_gather(sidx0, rows0, sem_g0)

            return carry

        lax.fori_loop(0, n_pairs, body, 0)
        wait_scatter(rows1, didx1, sem_s1)
        plsc.subcore_barrier()

        for kk in range((n_rw + NS - 1) // NS):
            k = s + NS * kk

            @pl.when(k < n_rw)
            def _dump():
                pltpu.sync_copy(acc.at[pl.ds(k * rw, rw)], stage)
                off = pl.multiple_of(k * rw, 8)
                pltpu.sync_copy(stage, out.at[c, pl.ds(off, rw)])

    return scat_k(g2f, src_both, dst)


_BN = 400


def _tc_first(x, w0, deg2):
    n, d = x.shape
    nb = n // _BN

    hw = d // 2

    def body(x_ref, w_ref, d0_ref, d1_ref, g_ref, dinv_ref):
        dinv = lax.rsqrt(d0_ref[:, 0:1] + d1_ref[:, 0:1] + 1.0)
        g = dinv * jnp.dot(x_ref[...], w_ref[...],
                           preferred_element_type=jnp.float32)
        g_ref[0] = g[:, :hw]
        g_ref[1] = g[:, hw:]
        dinv_ref[...] = dinv

    return pl.pallas_call(
        body,
        grid=(nb,),
        in_specs=[
            pl.BlockSpec((_BN, d), lambda i: (i, 0)),
            pl.BlockSpec((d, d), lambda i: (0, 0)),
            pl.BlockSpec((_BN, d), lambda i: (i, 0)),
            pl.BlockSpec((_BN, d), lambda i: (i + nb, 0)),
        ],
        out_specs=[
            pl.BlockSpec((2, _BN, hw), lambda i: (0, i, 0)),
            pl.BlockSpec((_BN, 1), lambda i: (i, 0)),
        ],
        out_shape=[
            jax.ShapeDtypeStruct((2, n, hw), jnp.float32),
            jax.ShapeDtypeStruct((n, 1), jnp.float32),
        ],
    )(x, w0, deg2, deg2)


def _tc_mid(s2, g2, dinv, b, w_next, lw, y):
    _, n, hw = g2.shape
    d = 2 * hw
    nb = n // _BN
    out = lw.shape[1]

    def body(s2_ref, g2_ref, dinv_ref, b_ref, w_ref, lw_ref, y_ref,
             gout_ref, yout_ref):
        dinv = dinv_ref[...]
        ss = jnp.concatenate([s2_ref[0], s2_ref[1]], axis=1)
        gg = jnp.concatenate([g2_ref[0], g2_ref[1]], axis=1)
        hh = jnp.maximum(dinv * (ss + gg) + b_ref[...], 0.0)
        yout_ref[...] = y_ref[...] + jnp.dot(hh, lw_ref[...],
                                             preferred_element_type=jnp.float32)
        gn = dinv * jnp.dot(hh, w_ref[...], preferred_element_type=jnp.float32)
        gout_ref[0] = gn[:, :hw]
        gout_ref[1] = gn[:, hw:]

    return pl.pallas_call(
        body,
        grid=(nb,),
        in_specs=[
            pl.BlockSpec((2, _BN, hw), lambda i: (0, i, 0)),
            pl.BlockSpec((2, _BN, hw), lambda i: (0, i, 0)),
            pl.BlockSpec((_BN, 1), lambda i: (i, 0)),
            pl.BlockSpec((1, d), lambda i: (0, 0)),
            pl.BlockSpec((d, d), lambda i: (0, 0)),
            pl.BlockSpec((d, out), lambda i: (0, 0)),
            pl.BlockSpec((_BN, out), lambda i: (i, 0)),
        ],
        out_specs=[
            pl.BlockSpec((2, _BN, hw), lambda i: (0, i, 0)),
            pl.BlockSpec((_BN, out), lambda i: (i, 0)),
        ],
        out_shape=[
            jax.ShapeDtypeStruct((2, n, hw), jnp.float32),
            jax.ShapeDtypeStruct((n, out), jnp.float32),
        ],
    )(s2, g2, dinv, b, w_next, lw, y)


def _tc_last(s2, g2, dinv, b, lw, linb, y):
    _, n, hw = g2.shape
    d = 2 * hw
    nb = n // _BN
    out = lw.shape[1]

    def body(s2_ref, g2_ref, dinv_ref, b_ref, lw_ref, lb_ref, y_ref,
             o_ref):
        dinv = dinv_ref[...]
        ss = jnp.concatenate([s2_ref[0], s2_ref[1]], axis=1)
        gg = jnp.concatenate([g2_ref[0], g2_ref[1]], axis=1)
        hh = jnp.maximum(dinv * (ss + gg) + b_ref[...], 0.0)
        o_ref[...] = (y_ref[...] + lb_ref[...]
                      + jnp.dot(hh, lw_ref[...],
                                preferred_element_type=jnp.float32))

    return pl.pallas_call(
        body,
        grid=(nb,),
        in_specs=[
            pl.BlockSpec((2, _BN, hw), lambda i: (0, i, 0)),
            pl.BlockSpec((2, _BN, hw), lambda i: (0, i, 0)),
            pl.BlockSpec((_BN, 1), lambda i: (i, 0)),
            pl.BlockSpec((1, d), lambda i: (0, 0)),
            pl.BlockSpec((d, out), lambda i: (0, 0)),
            pl.BlockSpec((1, out), lambda i: (0, 0)),
            pl.BlockSpec((_BN, out), lambda i: (i, 0)),
        ],
        out_specs=pl.BlockSpec((_BN, out), lambda i: (i, 0)),
        out_shape=jax.ShapeDtypeStruct((n, out), jnp.float32),
    )(s2, g2, dinv, b, lw, linb, y)


def kernel(x, edge_index, Ws, bs, linW, linb):
    n, d = x.shape
    e = edge_index.shape[1]
    l_layers, h, _ = Ws.shape
    out_w = linW.shape[1]

    src = edge_index[0].astype(jnp.int32)
    dst = edge_index[1].astype(jnp.int32)

    deg2 = _sc_degree(dst, n, h, 160)
    g2, dinv = _tc_first(x, Ws[0], deg2)
    y = jnp.zeros((n, out_w), jnp.float32)

    ch = 512
    for l in range(l_layers):
        s2 = _sc_aggregate(g2, src, dst, ch)
        b_l = bs[l].reshape(1, h)
        lw_l = linW[l * h:(l + 1) * h]
        if l < l_layers - 1:
            g2, y = _tc_mid(s2, g2, dinv, b_l, Ws[l + 1], lw_l, y)
        else:
            return _tc_last(s2, g2, dinv, b_l, lw_l, linb.reshape(1, out_w), y)


# speedup vs baseline: 1.0608x; 1.0608x over previous
import functools

import jax
import jax.numpy as jnp
from jax import lax
from jax.experimental import pallas as pl
from jax.experimental.pallas import tpu as pltpu
from jax.experimental.pallas import tpu_sc as plsc

NS = 16
NC = 2


def _sc_degree(dst, n, h, ch):
    e = dst.shape[0]
    round_e = NC * NS * ch
    n_chunks = -(-e // round_e)
    n_chunks += n_chunks % 2
    n_pairs = n_chunks // 2
    epad = round_e * n_chunks
    pad = epad - e
    if pad:
        fill = jnp.arange(pad, dtype=jnp.int32)
        dst = jnp.concatenate([dst, n + (fill % 8)])
    npad = n + 8
    per_tile = epad // (NC * NS)
    rw = 80
    n_rw = n // rw
    mesh = plsc.VectorSubcoreMesh(core_axis_name="c", subcore_axis_name="s")

    @functools.partial(
        pl.kernel,
        out_type=jax.ShapeDtypeStruct((NC * n, h), jnp.float32),
        mesh=mesh,
        scratch_types=[
            pltpu.VMEM((ch,), jnp.int32),
            pltpu.VMEM((ch,), jnp.int32),
            pltpu.VMEM((ch, h), jnp.float32),
            pltpu.VMEM((rw, h), jnp.float32),
            pltpu.VMEM_SHARED((npad, h), jnp.float32),
            pltpu.SemaphoreType.DMA,
            pltpu.SemaphoreType.DMA,
        ],
    )
    def deg_k(dst_hbm, out, didx0, didx1, ones_v, stage, acc,
              sem_i0, sem_i1):
        c = lax.axis_index("c")
        s = lax.axis_index("s")

        def fill_ones(i, carry):
            for j in range(h // 16):
                ones_v[i, pl.ds(j * 16, 16)] = jnp.ones((16,), jnp.float32)
            return carry

        lax.fori_loop(0, ch, fill_ones, 0)

        def fill_zeros(i, carry):
            for j in range(h // 16):
                stage[i, pl.ds(j * 16, 16)] = jnp.zeros((16,), jnp.float32)
            return carry

        lax.fori_loop(0, rw, fill_zeros, 0)
        for kk in range((n_rw + NS - 1) // NS):
            k = s + NS * kk

            @pl.when(k < n_rw)
            def _zero():
                pltpu.sync_copy(stage, acc.at[pl.ds(k * rw, rw)])

        plsc.subcore_barrier()
        base = (c * NS + s) * per_tile

        def eoff(ck):
            return pl.ds(base + ck * ch, ch)

        pltpu.async_copy(dst_hbm.at[eoff(0)], didx0, sem_i0)

        def body(t, carry):
            pltpu.async_copy(dst_hbm.at[eoff(2 * t + 1)], didx1, sem_i1)
            pltpu.make_async_copy(dst_hbm.at[eoff(0)], didx0, sem_i0).wait()
            pltpu.sync_copy(ones_v, acc.at[didx0], add=True)

            @pl.when(t < n_pairs - 1)
            def _prefetch0():
                pltpu.async_copy(dst_hbm.at[eoff(2 * t + 2)], didx0, sem_i0)

            pltpu.make_async_copy(dst_hbm.at[eoff(0)], didx1, sem_i1).wait()
            pltpu.sync_copy(ones_v, acc.at[didx1], add=True)
            return carry

        lax.fori_loop(0, n_pairs, body, 0)
        plsc.subcore_barrier()
        for kk in range((n_rw + NS - 1) // NS):
            k = s + NS * kk

            @pl.when(k < n_rw)
            def _dump():
                pltpu.sync_copy(acc.at[pl.ds(k * rw, rw)], stage)
                off = pl.multiple_of(c * n + k * rw, 8)
                pltpu.sync_copy(stage, out.at[pl.ds(off, rw)])

    return deg_k(dst)


def _sc_aggregate(g, src, dst, ch):
    n, h = g.shape
    e = src.shape[0]
    round_e = NC * NS * ch
    n_chunks = -(-e // round_e)
    n_chunks += n_chunks % 2
    n_pairs = n_chunks // 2
    epad = round_e * n_chunks
    pad = epad - e
    if pad:
        fill = jnp.arange(pad, dtype=jnp.int32)
        src = jnp.concatenate([src, fill % n])
        dst = jnp.concatenate([dst, n + (fill % 8)])
    npad = n + 8
    per_tile = epad // (NC * NS)
    rw = 80
    n_rw = n // rw
    mesh = plsc.VectorSubcoreMesh(core_axis_name="c", subcore_axis_name="s")

    @functools.partial(
        pl.kernel,
        out_type=jax.ShapeDtypeStruct((NC * n, h), jnp.float32),
        mesh=mesh,
        scratch_types=[
            pltpu.VMEM((ch,), jnp.int32),
            pltpu.VMEM((ch,), jnp.int32),
            pltpu.VMEM((ch,), jnp.int32),
            pltpu.VMEM((ch,), jnp.int32),
            pltpu.VMEM((ch, h), jnp.float32),
            pltpu.VMEM((ch, h), jnp.float32),
            pltpu.VMEM_SHARED((npad, h), jnp.float32),
            pltpu.SemaphoreType.DMA,
            pltpu.SemaphoreType.DMA,
            pltpu.SemaphoreType.DMA,
            pltpu.SemaphoreType.DMA,
            pltpu.SemaphoreType.DMA,
            pltpu.SemaphoreType.DMA,
        ],
    )
    def scat_k(g_hbm, src_hbm, dst_hbm, out,
               sidx0, didx0, sidx1, didx1, rows0, rows1, acc,
               sem_g0, sem_g1, sem_i0, sem_i1, sem_s0, sem_s1):
        c = lax.axis_index("c")
        s = lax.axis_index("s")

        def zbody(i, carry):
            for j in range(h // 16):
                rows0[i, pl.ds(j * 16, 16)] = jnp.zeros((16,), jnp.float32)
            return carry

        lax.fori_loop(0, rw, zbody, 0)
        stage = rows0.at[pl.ds(0, rw)]
        for kk in range((n_rw + NS - 1) // NS):
            k = s + NS * kk

            @pl.when(k < n_rw)
            def _zero():
                pltpu.sync_copy(stage, acc.at[pl.ds(k * rw, rw)])

        plsc.subcore_barrier()
        base = (c * NS + s) * per_tile

        def eoff(ck):
            return pl.ds(base + ck * ch, ch)

        def start_idx(ck, sidx, didx, sem):
            pltpu.async_copy(src_hbm.at[eoff(ck)], sidx, sem)
            pltpu.async_copy(dst_hbm.at[eoff(ck)], didx, sem)

        def wait_idx(ck, sidx, didx, sem):
            pltpu.make_async_copy(src_hbm.at[eoff(ck)], sidx, sem).wait()
            pltpu.make_async_copy(dst_hbm.at[eoff(ck)], didx, sem).wait()

        def start_gather(sidx, rows, sem):
            pltpu.async_copy(g_hbm.at[sidx], rows, sem)

        def wait_gather(sidx, rows, sem):
            pltpu.make_async_copy(g_hbm.at[sidx], rows, sem).wait()

        def start_scatter(rows, didx, sem):
            pltpu.async_copy(rows, acc.at[didx], sem, add=True)

        def wait_scatter(rows, didx, sem):
            pltpu.make_async_copy(rows, acc.at[didx], sem).wait()

        start_idx(0, sidx0, didx0, sem_i0)
        wait_idx(0, sidx0, didx0, sem_i0)
        start_gather(sidx0, rows0, sem_g0)

        def body(t, carry):
            @pl.when(t > 0)
            def _drain1():
                wait_scatter(rows1, didx1, sem_s1)

            start_idx(2 * t + 1, sidx1, didx1, sem_i1)
            wait_gather(sidx0, rows0, sem_g0)
            start_scatter(rows0, didx0, sem_s0)
            wait_idx(2 * t + 1, sidx1, didx1, sem_i1)
            start_gather(sidx1, rows1, sem_g1)
            wait_scatter(rows0, didx0, sem_s0)

            @pl.when(t < n_pairs - 1)
            def _prefetch0():
                start_idx(2 * t + 2, sidx0, didx0, sem_i0)

            wait_gather(sidx1, rows1, sem_g1)
            start_scatter(rows1, didx1, sem_s1)

            @pl.when(t < n_pairs - 1)
            def _gather0():
                wait_idx(2 * t + 2, sidx0, didx0, sem_i0)
                start_gather(sidx0, rows0, sem_g0)

            return carry

        lax.fori_loop(0, n_pairs, body, 0)
        wait_scatter(rows1, didx1, sem_s1)
        plsc.subcore_barrier()

        for kk in range((n_rw + NS - 1) // NS):
            k = s + NS * kk

            @pl.when(k < n_rw)
            def _dump():
                pltpu.sync_copy(acc.at[pl.ds(k * rw, rw)], stage)
                off = pl.multiple_of(c * n + k * rw, 8)
                pltpu.sync_copy(stage, out.at[pl.ds(off, rw)])

    return scat_k(g, src, dst)


_BN = 400


def _tc_first(x, w0, deg2):
    n, d = x.shape
    nb = n // _BN

    def body(x_ref, w_ref, d0_ref, d1_ref, g_ref, dinv_ref):
        dinv = lax.rsqrt(d0_ref[:, 0:1] + d1_ref[:, 0:1] + 1.0)
        g_ref[...] = dinv * jnp.dot(x_ref[...], w_ref[...],
                                    preferred_element_type=jnp.float32)
        dinv_ref[...] = dinv

    return pl.pallas_call(
        body,
        grid=(nb,),
        in_specs=[
            pl.BlockSpec((_BN, d), lambda i: (i, 0)),
            pl.BlockSpec((d, d), lambda i: (0, 0)),
            pl.BlockSpec((_BN, d), lambda i: (i, 0)),
            pl.BlockSpec((_BN, d), lambda i: (i + nb, 0)),
        ],
        out_specs=[
            pl.BlockSpec((_BN, d), lambda i: (i, 0)),
            pl.BlockSpec((_BN, 1), lambda i: (i, 0)),
        ],
        out_shape=[
            jax.ShapeDtypeStruct((n, d), jnp.float32),
            jax.ShapeDtypeStruct((n, 1), jnp.float32),
        ],
    )(x, w0, deg2, deg2)


def _tc_mid(s2, g, dinv, b, w_next, lw, y):
    n, d = g.shape
    nb = n // _BN
    out = lw.shape[1]

    def body(s0_ref, s1_ref, g_ref, dinv_ref, b_ref, w_ref, lw_ref, y_ref,
             gout_ref, yout_ref):
        dinv = dinv_ref[...]
        hh = jnp.maximum(dinv * (s0_ref[...] + s1_ref[...] + g_ref[...])
                         + b_ref[...], 0.0)
        yout_ref[...] = y_ref[...] + jnp.dot(hh, lw_ref[...],
                                             preferred_element_type=jnp.float32)
        gout_ref[...] = dinv * jnp.dot(hh, w_ref[...],
                                       preferred_element_type=jnp.float32)

    return pl.pallas_call(
        body,
        grid=(nb,),
        in_specs=[
            pl.BlockSpec((_BN, d), lambda i: (i, 0)),
            pl.BlockSpec((_BN, d), lambda i: (i + nb, 0)),
            pl.BlockSpec((_BN, d), lambda i: (i, 0)),
            pl.BlockSpec((_BN, 1), lambda i: (i, 0)),
            pl.BlockSpec((1, d), lambda i: (0, 0)),
            pl.BlockSpec((d, d), lambda i: (0, 0)),
            pl.BlockSpec((d, out), lambda i: (0, 0)),
            pl.BlockSpec((_BN, out), lambda i: (i, 0)),
        ],
        out_specs=[
            pl.BlockSpec((_BN, d), lambda i: (i, 0)),
            pl.BlockSpec((_BN, out), lambda i: (i, 0)),
        ],
        out_shape=[
            jax.ShapeDtypeStruct((n, d), jnp.float32),
            jax.ShapeDtypeStruct((n, out), jnp.float32),
        ],
    )(s2, s2, g, dinv, b, w_next, lw, y)


def _tc_last(s2, g, dinv, b, lw, linb, y):
    n, d = g.shape
    nb = n // _BN
    out = lw.shape[1]

    def body(s0_ref, s1_ref, g_ref, dinv_ref, b_ref, lw_ref, lb_ref, y_ref,
             o_ref):
        dinv = dinv_ref[...]
        hh = jnp.maximum(dinv * (s0_ref[...] + s1_ref[...] + g_ref[...])
                         + b_ref[...], 0.0)
        o_ref[...] = (y_ref[...] + lb_ref[...]
                      + jnp.dot(hh, lw_ref[...],
                                preferred_element_type=jnp.float32))

    return pl.pallas_call(
        body,
        grid=(nb,),
        in_specs=[
            pl.BlockSpec((_BN, d), lambda i: (i, 0)),
            pl.BlockSpec((_BN, d), lambda i: (i + nb, 0)),
            pl.BlockSpec((_BN, d), lambda i: (i, 0)),
            pl.BlockSpec((_BN, 1), lambda i: (i, 0)),
            pl.BlockSpec((1, d), lambda i: (0, 0)),
            pl.BlockSpec((d, out), lambda i: (0, 0)),
            pl.BlockSpec((1, out), lambda i: (0, 0)),
            pl.BlockSpec((_BN, out), lambda i: (i, 0)),
        ],
        out_specs=pl.BlockSpec((_BN, out), lambda i: (i, 0)),
        out_shape=jax.ShapeDtypeStruct((n, out), jnp.float32),
    )(s2, s2, g, dinv, b, lw, linb, y)


def kernel(x, edge_index, Ws, bs, linW, linb):
    n, d = x.shape
    e = edge_index.shape[1]
    l_layers, h, _ = Ws.shape
    out_w = linW.shape[1]

    src = edge_index[0].astype(jnp.int32)
    dst = edge_index[1].astype(jnp.int32)

    deg2 = _sc_degree(dst, n, h, 160)
    g, dinv = _tc_first(x, Ws[0], deg2)
    y = jnp.zeros((n, out_w), jnp.float32)

    ch = 160
    for l in range(l_layers):
        s2 = _sc_aggregate(g, src, dst, ch)
        b_l = bs[l].reshape(1, h)
        lw_l = linW[l * h:(l + 1) * h]
        if l < l_layers - 1:
            g, y = _tc_mid(s2, g, dinv, b_l, Ws[l + 1], lw_l, y)
        else:
            return _tc_last(s2, g, dinv, b_l, lw_l, linb.reshape(1, out_w), y)


# speedup vs baseline: 1.1256x; 1.0612x over previous
import functools

import jax
import jax.numpy as jnp
from jax import lax
from jax.experimental import pallas as pl
from jax.experimental.pallas import tpu as pltpu
from jax.experimental.pallas import tpu_sc as plsc

NS = 16
NC = 2


def _sc_degree(dst, n, h, ch):
    e = dst.shape[0]
    round_e = NC * NS * ch
    n_chunks = -(-e // round_e)
    n_chunks += n_chunks % 2
    n_pairs = n_chunks // 2
    epad = round_e * n_chunks
    pad = epad - e
    if pad:
        fill = jnp.arange(pad, dtype=jnp.int32)
        dst = jnp.concatenate([dst, n + (fill % 8)])
    npad = n + 8
    per_tile = epad // (NC * NS)
    rw = 80
    n_rw = n // rw
    mesh = plsc.VectorSubcoreMesh(core_axis_name="c", subcore_axis_name="s")

    @functools.partial(
        pl.kernel,
        out_type=jax.ShapeDtypeStruct((NC * n, h), jnp.float32),
        mesh=mesh,
        scratch_types=[
            pltpu.VMEM((ch,), jnp.int32),
            pltpu.VMEM((ch,), jnp.int32),
            pltpu.VMEM((ch, h), jnp.float32),
            pltpu.VMEM((rw, h), jnp.float32),
            pltpu.VMEM_SHARED((npad, h), jnp.float32),
            pltpu.SemaphoreType.DMA,
            pltpu.SemaphoreType.DMA,
        ],
    )
    def deg_k(dst_hbm, out, didx0, didx1, ones_v, stage, acc,
              sem_i0, sem_i1):
        c = lax.axis_index("c")
        s = lax.axis_index("s")

        def fill_ones(i, carry):
            for j in range(h // 16):
                ones_v[i, pl.ds(j * 16, 16)] = jnp.ones((16,), jnp.float32)
            return carry

        lax.fori_loop(0, ch, fill_ones, 0)

        def fill_zeros(i, carry):
            for j in range(h // 16):
                stage[i, pl.ds(j * 16, 16)] = jnp.zeros((16,), jnp.float32)
            return carry

        lax.fori_loop(0, rw, fill_zeros, 0)
        for kk in range((n_rw + NS - 1) // NS):
            k = s + NS * kk

            @pl.when(k < n_rw)
            def _zero():
                pltpu.sync_copy(stage, acc.at[pl.ds(k * rw, rw)])

        plsc.subcore_barrier()
        base = (c * NS + s) * per_tile

        def eoff(ck):
            return pl.ds(base + ck * ch, ch)

        pltpu.async_copy(dst_hbm.at[eoff(0)], didx0, sem_i0)

        def body(t, carry):
            pltpu.async_copy(dst_hbm.at[eoff(2 * t + 1)], didx1, sem_i1)
            pltpu.make_async_copy(dst_hbm.at[eoff(0)], didx0, sem_i0).wait()
            pltpu.sync_copy(ones_v, acc.at[didx0], add=True)

            @pl.when(t < n_pairs - 1)
            def _prefetch0():
                pltpu.async_copy(dst_hbm.at[eoff(2 * t + 2)], didx0, sem_i0)

            pltpu.make_async_copy(dst_hbm.at[eoff(0)], didx1, sem_i1).wait()
            pltpu.sync_copy(ones_v, acc.at[didx1], add=True)
            return carry

        lax.fori_loop(0, n_pairs, body, 0)
        plsc.subcore_barrier()
        for kk in range((n_rw + NS - 1) // NS):
            k = s + NS * kk

            @pl.when(k < n_rw)
            def _dump():
                pltpu.sync_copy(acc.at[pl.ds(k * rw, rw)], stage)
                off = pl.multiple_of(c * n + k * rw, 8)
                pltpu.sync_copy(stage, out.at[pl.ds(off, rw)])

    return deg_k(dst)


def _sc_aggregate(g, src, dst, ch):
    n, h = g.shape
    e = src.shape[0]
    round_e = NC * NS * ch
    n_chunks = -(-e // round_e)
    n_chunks += n_chunks % 2
    n_pairs = n_chunks // 2
    epad = round_e * n_chunks
    pad = epad - e
    if pad:
        fill = jnp.arange(pad, dtype=jnp.int32)
        src = jnp.concatenate([src, fill % n])
        dst = jnp.concatenate([dst, n + (fill % 8)])
    npad = n + 8
    per_tile = epad // (NC * NS)
    rw = 80
    n_rw = n // rw
    mesh = plsc.VectorSubcoreMesh(core_axis_name="c", subcore_axis_name="s")

    @functools.partial(
        pl.kernel,
        out_type=jax.ShapeDtypeStruct((NC * n, h), jnp.float32),
        mesh=mesh,
        scratch_types=[
            pltpu.VMEM((ch,), jnp.int32),
            pltpu.VMEM((ch,), jnp.int32),
            pltpu.VMEM((ch,), jnp.int32),
            pltpu.VMEM((ch,), jnp.int32),
            pltpu.VMEM((ch, h), jnp.float32),
            pltpu.VMEM((ch, h), jnp.float32),
            pltpu.VMEM_SHARED((npad, h), jnp.float32),
            pltpu.SemaphoreType.DMA,
            pltpu.SemaphoreType.DMA,
            pltpu.SemaphoreType.DMA,
            pltpu.SemaphoreType.DMA,
            pltpu.SemaphoreType.DMA,
            pltpu.SemaphoreType.DMA,
        ],
    )
    def scat_k(g_hbm, src_hbm, dst_hbm, out,
               sidx0, didx0, sidx1, didx1, rows0, rows1, acc,
               sem_g0, sem_g1, sem_i0, sem_i1, sem_s0, sem_s1):
        c = lax.axis_index("c")
        s = lax.axis_index("s")

        def zbody(i, carry):
            for j in range(h // 16):
                rows0[i, pl.ds(j * 16, 16)] = jnp.zeros((16,), jnp.float32)
            return carry

        lax.fori_loop(0, rw, zbody, 0)
        stage = rows0.at[pl.ds(0, rw)]
        for kk in range((n_rw + NS - 1) // NS):
            k = s + NS * kk

            @pl.when(k < n_rw)
            def _zero():
                pltpu.sync_copy(stage, acc.at[pl.ds(k * rw, rw)])

        plsc.subcore_barrier()
        base = (c * NS + s) * per_tile

        def eoff(ck):
            return pl.ds(base + ck * ch, ch)

        def start_idx(ck, sidx, didx, sem):
            pltpu.async_copy(src_hbm.at[eoff(ck)], sidx, sem)
            pltpu.async_copy(dst_hbm.at[eoff(ck)], didx, sem)

        def wait_idx(ck, sidx, didx, sem):
            pltpu.make_async_copy(src_hbm.at[eoff(ck)], sidx, sem).wait()
            pltpu.make_async_copy(dst_hbm.at[eoff(ck)], didx, sem).wait()

        def start_gather(sidx, rows, sem):
            pltpu.async_copy(g_hbm.at[sidx], rows, sem)

        def wait_gather(sidx, rows, sem):
            pltpu.make_async_copy(g_hbm.at[sidx], rows, sem).wait()

        def start_scatter(rows, didx, sem):
            pltpu.async_copy(rows, acc.at[didx], sem, add=True)

        def wait_scatter(rows, didx, sem):
            pltpu.make_async_copy(rows, acc.at[didx], sem).wait()

        start_idx(0, sidx0, didx0, sem_i0)
        wait_idx(0, sidx0, didx0, sem_i0)
        start_gather(sidx0, rows0, sem_g0)

        def body(t, carry):
            @pl.when(t > 0)
            def _drain1():
                wait_scatter(rows1, didx1, sem_s1)

            start_idx(2 * t + 1, sidx1, didx1, sem_i1)
            wait_gather(sidx0, rows0, sem_g0)
            start_scatter(rows0, didx0, sem_s0)
            wait_idx(2 * t + 1, sidx1, didx1, sem_i1)
            start_gather(sidx1, rows1, sem_g1)
            wait_scatter(rows0, didx0, sem_s0)

            @pl.when(t < n_pairs - 1)
            def _prefetch0():
                start_idx(2 * t + 2, sidx0, didx0, sem_i0)

            wait_gather(sidx1, rows1, sem_g1)
            start_scatter(rows1, didx1, sem_s1)

            @pl.when(t < n_pairs - 1)
            def _gather0():
                wait_idx(2 * t + 2, sidx0, didx0, sem_i0)
                start_gather(sidx0, rows0, sem_g0)

            return carry

        lax.fori_loop(0, n_pairs, body, 0)
        wait_scatter(rows1, didx1, sem_s1)
        plsc.subcore_barrier()

        for kk in range((n_rw + NS - 1) // NS):
            k = s + NS * kk

            @pl.when(k < n_rw)
            def _dump():
                pltpu.sync_copy(acc.at[pl.ds(k * rw, rw)], stage)
                off = pl.multiple_of(c * n + k * rw, 8)
                pltpu.sync_copy(stage, out.at[pl.ds(off, rw)])

    return scat_k(g, src, dst)


_BN = 1000


def _tc_first(x, w0, deg2):
    n, d = x.shape
    nb = n // _BN

    def body(x_ref, w_ref, d0_ref, d1_ref, g_ref, dinv_ref):
        dinv = lax.rsqrt(d0_ref[:, 0:1] + d1_ref[:, 0:1] + 1.0)
        g_ref[...] = dinv * jnp.dot(x_ref[...], w_ref[...],
                                    preferred_element_type=jnp.float32)
        dinv_ref[...] = dinv

    return pl.pallas_call(
        body,
        grid=(nb,),
        in_specs=[
            pl.BlockSpec((_BN, d), lambda i: (i, 0)),
            pl.BlockSpec((d, d), lambda i: (0, 0)),
            pl.BlockSpec((_BN, d), lambda i: (i, 0)),
            pl.BlockSpec((_BN, d), lambda i: (i + nb, 0)),
        ],
        out_specs=[
            pl.BlockSpec((_BN, d), lambda i: (i, 0)),
            pl.BlockSpec((_BN, 1), lambda i: (i, 0)),
        ],
        out_shape=[
            jax.ShapeDtypeStruct((n, d), jnp.float32),
            jax.ShapeDtypeStruct((n, 1), jnp.float32),
        ],
    )(x, w0, deg2, deg2)


def _tc_mid(s2, g, dinv, b, w_next, lw, y):
    n, d = g.shape
    nb = n // _BN
    out = lw.shape[1]

    def body(s0_ref, s1_ref, g_ref, dinv_ref, b_ref, w_ref, lw_ref, y_ref,
             gout_ref, yout_ref):
        dinv = dinv_ref[...]
        hh = jnp.maximum(dinv * (s0_ref[...] + s1_ref[...] + g_ref[...])
                         + b_ref[...], 0.0)
        yout_ref[...] = y_ref[...] + jnp.dot(hh, lw_ref[...],
                                             preferred_element_type=jnp.float32)
        gout_ref[...] = dinv * jnp.dot(hh, w_ref[...],
                                       preferred_element_type=jnp.float32)

    return pl.pallas_call(
        body,
        grid=(nb,),
        in_specs=[
            pl.BlockSpec((_BN, d), lambda i: (i, 0)),
            pl.BlockSpec((_BN, d), lambda i: (i + nb, 0)),
            pl.BlockSpec((_BN, d), lambda i: (i, 0)),
            pl.BlockSpec((_BN, 1), lambda i: (i, 0)),
            pl.BlockSpec((1, d), lambda i: (0, 0)),
            pl.BlockSpec((d, d), lambda i: (0, 0)),
            pl.BlockSpec((d, out), lambda i: (0, 0)),
            pl.BlockSpec((_BN, out), lambda i: (i, 0)),
        ],
        out_specs=[
            pl.BlockSpec((_BN, d), lambda i: (i, 0)),
            pl.BlockSpec((_BN, out), lambda i: (i, 0)),
        ],
        out_shape=[
            jax.ShapeDtypeStruct((n, d), jnp.float32),
            jax.ShapeDtypeStruct((n, out), jnp.float32),
        ],
    )(s2, s2, g, dinv, b, w_next, lw, y)


def _tc_last(s2, g, dinv, b, lw, linb, y):
    n, d = g.shape
    nb = n // _BN
    out = lw.shape[1]

    def body(s0_ref, s1_ref, g_ref, dinv_ref, b_ref, lw_ref, lb_ref, y_ref,
             o_ref):
        dinv = dinv_ref[...]
        hh = jnp.maximum(dinv * (s0_ref[...] + s1_ref[...] + g_ref[...])
                         + b_ref[...], 0.0)
        o_ref[...] = (y_ref[...] + lb_ref[...]
                      + jnp.dot(hh, lw_ref[...],
                                preferred_element_type=jnp.float32))

    return pl.pallas_call(
        body,
        grid=(nb,),
        in_specs=[
            pl.BlockSpec((_BN, d), lambda i: (i, 0)),
            pl.BlockSpec((_BN, d), lambda i: (i + nb, 0)),
            pl.BlockSpec((_BN, d), lambda i: (i, 0)),
            pl.BlockSpec((_BN, 1), lambda i: (i, 0)),
            pl.BlockSpec((1, d), lambda i: (0, 0)),
            pl.BlockSpec((d, out), lambda i: (0, 0)),
            pl.BlockSpec((1, out), lambda i: (0, 0)),
            pl.BlockSpec((_BN, out), lambda i: (i, 0)),
        ],
        out_specs=pl.BlockSpec((_BN, out), lambda i: (i, 0)),
        out_shape=jax.ShapeDtypeStruct((n, out), jnp.float32),
    )(s2, s2, g, dinv, b, lw, linb, y)


def kernel(x, edge_index, Ws, bs, linW, linb):
    n, d = x.shape
    e = edge_index.shape[1]
    l_layers, h, _ = Ws.shape
    out_w = linW.shape[1]

    src = edge_index[0].astype(jnp.int32)
    dst = edge_index[1].astype(jnp.int32)

    deg2 = _sc_degree(dst, n, h, 160)
    g, dinv = _tc_first(x, Ws[0], deg2)
    y = jnp.zeros((n, out_w), jnp.float32)

    ch = 160
    for l in range(l_layers):
        s2 = _sc_aggregate(g, src, dst, ch)
        b_l = bs[l].reshape(1, h)
        lw_l = linW[l * h:(l + 1) * h]
        if l < l_layers - 1:
            g, y = _tc_mid(s2, g, dinv, b_l, Ws[l + 1], lw_l, y)
        else:
            return _tc_last(s2, g, dinv, b_l, lw_l, linb.reshape(1, out_w), y)


# speedup vs baseline: 1.1418x; 1.0144x over previous
import functools

import jax
import jax.numpy as jnp
from jax import lax
from jax.experimental import pallas as pl
from jax.experimental.pallas import tpu as pltpu
from jax.experimental.pallas import tpu_sc as plsc

NS = 16
NC = 2


def _sc_degree(dst, n, h, ch):
    e = dst.shape[0]
    round_e = NC * NS * ch
    n_chunks = -(-e // round_e)
    n_chunks += n_chunks % 2
    n_pairs = n_chunks // 2
    epad = round_e * n_chunks
    pad = epad - e
    if pad:
        fill = jnp.arange(pad, dtype=jnp.int32)
        dst = jnp.concatenate([dst, n + (fill % 8)])
    npad = n + 8
    per_tile = epad // (NC * NS)
    rw = 80
    n_rw = n // rw
    mesh = plsc.VectorSubcoreMesh(core_axis_name="c", subcore_axis_name="s")

    @functools.partial(
        pl.kernel,
        out_type=jax.ShapeDtypeStruct((NC * n, h), jnp.float32),
        mesh=mesh,
        scratch_types=[
            pltpu.VMEM((ch,), jnp.int32),
            pltpu.VMEM((ch,), jnp.int32),
            pltpu.VMEM((ch, h), jnp.float32),
            pltpu.VMEM((rw, h), jnp.float32),
            pltpu.VMEM_SHARED((npad, h), jnp.float32),
            pltpu.SemaphoreType.DMA,
            pltpu.SemaphoreType.DMA,
        ],
    )
    def deg_k(dst_hbm, out, didx0, didx1, ones_v, stage, acc,
              sem_i0, sem_i1):
        c = lax.axis_index("c")
        s = lax.axis_index("s")

        def fill_ones(i, carry):
            for j in range(h // 16):
                ones_v[i, pl.ds(j * 16, 16)] = jnp.ones((16,), jnp.float32)
            return carry

        lax.fori_loop(0, ch, fill_ones, 0)

        def fill_zeros(i, carry):
            for j in range(h // 16):
                stage[i, pl.ds(j * 16, 16)] = jnp.zeros((16,), jnp.float32)
            return carry

        lax.fori_loop(0, rw, fill_zeros, 0)
        for kk in range((n_rw + NS - 1) // NS):
            k = s + NS * kk

            @pl.when(k < n_rw)
            def _zero():
                pltpu.sync_copy(stage, acc.at[pl.ds(k * rw, rw)])

        plsc.subcore_barrier()
        base = (c * NS + s) * per_tile

        def eoff(ck):
            return pl.ds(base + ck * ch, ch)

        pltpu.async_copy(dst_hbm.at[eoff(0)], didx0, sem_i0)

        def body(t, carry):
            pltpu.async_copy(dst_hbm.at[eoff(2 * t + 1)], didx1, sem_i1)
            pltpu.make_async_copy(dst_hbm.at[eoff(0)], didx0, sem_i0).wait()
            pltpu.sync_copy(ones_v, acc.at[didx0], add=True)

            @pl.when(t < n_pairs - 1)
            def _prefetch0():
                pltpu.async_copy(dst_hbm.at[eoff(2 * t + 2)], didx0, sem_i0)

            pltpu.make_async_copy(dst_hbm.at[eoff(0)], didx1, sem_i1).wait()
            pltpu.sync_copy(ones_v, acc.at[didx1], add=True)
            return carry

        lax.fori_loop(0, n_pairs, body, 0)
        plsc.subcore_barrier()
        for kk in range((n_rw + NS - 1) // NS):
            k = s + NS * kk

            @pl.when(k < n_rw)
            def _dump():
                pltpu.sync_copy(acc.at[pl.ds(k * rw, rw)], stage)
                off = pl.multiple_of(c * n + k * rw, 8)
                pltpu.sync_copy(stage, out.at[pl.ds(off, rw)])

    return deg_k(dst)


def _sc_aggregate(g, src, dst, ch):
    n, h = g.shape
    e = src.shape[0]
    round_e = NC * NS * ch
    n_chunks = -(-e // round_e)
    n_chunks += n_chunks % 2
    n_pairs = n_chunks // 2
    epad = round_e * n_chunks
    pad = epad - e
    if pad:
        fill = jnp.arange(pad, dtype=jnp.int32)
        src = jnp.concatenate([src, fill % n])
        dst = jnp.concatenate([dst, n + (fill % 8)])
    npad = n + 8
    per_tile = epad // (NC * NS)
    rw = 80
    n_rw = n // rw
    mesh = plsc.VectorSubcoreMesh(core_axis_name="c", subcore_axis_name="s")

    @functools.partial(
        pl.kernel,
        out_type=jax.ShapeDtypeStruct((NC * n, h), jnp.float32),
        mesh=mesh,
        scratch_types=[
            pltpu.VMEM((ch,), jnp.int32),
            pltpu.VMEM((ch,), jnp.int32),
            pltpu.VMEM((ch,), jnp.int32),
            pltpu.VMEM((ch,), jnp.int32),
            pltpu.VMEM((ch, h), jnp.float32),
            pltpu.VMEM((ch, h), jnp.float32),
            pltpu.VMEM_SHARED((npad, h), jnp.float32),
            pltpu.SemaphoreType.DMA,
            pltpu.SemaphoreType.DMA,
            pltpu.SemaphoreType.DMA,
            pltpu.SemaphoreType.DMA,
            pltpu.SemaphoreType.DMA,
            pltpu.SemaphoreType.DMA,
        ],
    )
    def scat_k(g_hbm, src_hbm, dst_hbm, out,
               sidx0, didx0, sidx1, didx1, rows0, rows1, acc,
               sem_g0, sem_g1, sem_i0, sem_i1, sem_s0, sem_s1):
        c = lax.axis_index("c")
        s = lax.axis_index("s")

        def zbody(i, carry):
            for j in range(h // 16):
                rows0[i, pl.ds(j * 16, 16)] = jnp.zeros((16,), jnp.float32)
            return carry

        lax.fori_loop(0, rw, zbody, 0)
        stage = rows0.at[pl.ds(0, rw)]
        for kk in range((n_rw + NS - 1) // NS):
            k = s + NS * kk

            @pl.when(k < n_rw)
            def _zero():
                pltpu.sync_copy(stage, acc.at[pl.ds(k * rw, rw)])

        plsc.subcore_barrier()
        base = (c * NS + s) * per_tile

        def eoff(ck):
            return pl.ds(base + ck * ch, ch)

        def start_idx(ck, sidx, didx, sem):
            pltpu.async_copy(src_hbm.at[eoff(ck)], sidx, sem)
            pltpu.async_copy(dst_hbm.at[eoff(ck)], didx, sem)

        def wait_idx(ck, sidx, didx, sem):
            pltpu.make_async_copy(src_hbm.at[eoff(ck)], sidx, sem).wait()
            pltpu.make_async_copy(dst_hbm.at[eoff(ck)], didx, sem).wait()

        def start_gather(sidx, rows, sem):
            pltpu.async_copy(g_hbm.at[sidx], rows, sem)

        def wait_gather(sidx, rows, sem):
            pltpu.make_async_copy(g_hbm.at[sidx], rows, sem).wait()

        def start_scatter(rows, didx, sem):
            pltpu.async_copy(rows, acc.at[didx], sem, add=True)

        def wait_scatter(rows, didx, sem):
            pltpu.make_async_copy(rows, acc.at[didx], sem).wait()

        start_idx(0, sidx0, didx0, sem_i0)
        wait_idx(0, sidx0, didx0, sem_i0)
        start_gather(sidx0, rows0, sem_g0)

        def body(t, carry):
            @pl.when(t > 0)
            def _drain1():
                wait_scatter(rows1, didx1, sem_s1)

            start_idx(2 * t + 1, sidx1, didx1, sem_i1)
            wait_gather(sidx0, rows0, sem_g0)
            start_scatter(rows0, didx0, sem_s0)
            wait_idx(2 * t + 1, sidx1, didx1, sem_i1)
            start_gather(sidx1, rows1, sem_g1)
            wait_scatter(rows0, didx0, sem_s0)

            @pl.when(t < n_pairs - 1)
            def _prefetch0():
                start_idx(2 * t + 2, sidx0, didx0, sem_i0)

            wait_gather(sidx1, rows1, sem_g1)
            start_scatter(rows1, didx1, sem_s1)

            @pl.when(t < n_pairs - 1)
            def _gather0():
                wait_idx(2 * t + 2, sidx0, didx0, sem_i0)
                start_gather(sidx0, rows0, sem_g0)

            return carry

        lax.fori_loop(0, n_pairs, body, 0)
        wait_scatter(rows1, didx1, sem_s1)
        plsc.subcore_barrier()

        for kk in range((n_rw + NS - 1) // NS):
            k = s + NS * kk

            @pl.when(k < n_rw)
            def _dump():
                pltpu.sync_copy(acc.at[pl.ds(k * rw, rw)], stage)
                off = pl.multiple_of(c * n + k * rw, 8)
                pltpu.sync_copy(stage, out.at[pl.ds(off, rw)])

    return scat_k(g, src, dst)


_BN = 2000


def _tc_first(x, w0, deg2):
    n, d = x.shape
    nb = n // _BN

    def body(x_ref, w_ref, d0_ref, d1_ref, g_ref, dinv_ref):
        dinv = lax.rsqrt(d0_ref[:, 0:1] + d1_ref[:, 0:1] + 1.0)
        g_ref[...] = dinv * jnp.dot(x_ref[...], w_ref[...],
                                    preferred_element_type=jnp.float32)
        dinv_ref[...] = dinv

    return pl.pallas_call(
        body,
        grid=(nb,),
        in_specs=[
            pl.BlockSpec((_BN, d), lambda i: (i, 0)),
            pl.BlockSpec((d, d), lambda i: (0, 0)),
            pl.BlockSpec((_BN, d), lambda i: (i, 0)),
            pl.BlockSpec((_BN, d), lambda i: (i + nb, 0)),
        ],
        out_specs=[
            pl.BlockSpec((_BN, d), lambda i: (i, 0)),
            pl.BlockSpec((_BN, 1), lambda i: (i, 0)),
        ],
        out_shape=[
            jax.ShapeDtypeStruct((n, d), jnp.float32),
            jax.ShapeDtypeStruct((n, 1), jnp.float32),
        ],
    )(x, w0, deg2, deg2)


def _tc_mid(s2, g, dinv, b, w_next, lw, y):
    n, d = g.shape
    nb = n // _BN
    out = lw.shape[1]

    def body(s0_ref, s1_ref, g_ref, dinv_ref, b_ref, w_ref, lw_ref, y_ref,
             gout_ref, yout_ref):
        dinv = dinv_ref[...]
        hh = jnp.maximum(dinv * (s0_ref[...] + s1_ref[...] + g_ref[...])
                         + b_ref[...], 0.0)
        yout_ref[...] = y_ref[...] + jnp.dot(hh, lw_ref[...],
                                             preferred_element_type=jnp.float32)
        gout_ref[...] = dinv * jnp.dot(hh, w_ref[...],
                                       preferred_element_type=jnp.float32)

    return pl.pallas_call(
        body,
        grid=(nb,),
        in_specs=[
            pl.BlockSpec((_BN, d), lambda i: (i, 0)),
            pl.BlockSpec((_BN, d), lambda i: (i + nb, 0)),
            pl.BlockSpec((_BN, d), lambda i: (i, 0)),
            pl.BlockSpec((_BN, 1), lambda i: (i, 0)),
            pl.BlockSpec((1, d), lambda i: (0, 0)),
            pl.BlockSpec((d, d), lambda i: (0, 0)),
            pl.BlockSpec((d, out), lambda i: (0, 0)),
            pl.BlockSpec((_BN, out), lambda i: (i, 0)),
        ],
        out_specs=[
            pl.BlockSpec((_BN, d), lambda i: (i, 0)),
            pl.BlockSpec((_BN, out), lambda i: (i, 0)),
        ],
        out_shape=[
            jax.ShapeDtypeStruct((n, d), jnp.float32),
            jax.ShapeDtypeStruct((n, out), jnp.float32),
        ],
    )(s2, s2, g, dinv, b, w_next, lw, y)


def _tc_last(s2, g, dinv, b, lw, linb, y):
    n, d = g.shape
    nb = n // _BN
    out = lw.shape[1]

    def body(s0_ref, s1_ref, g_ref, dinv_ref, b_ref, lw_ref, lb_ref, y_ref,
             o_ref):
        dinv = dinv_ref[...]
        hh = jnp.maximum(dinv * (s0_ref[...] + s1_ref[...] + g_ref[...])
                         + b_ref[...], 0.0)
        o_ref[...] = (y_ref[...] + lb_ref[...]
                      + jnp.dot(hh, lw_ref[...],
                                preferred_element_type=jnp.float32))

    return pl.pallas_call(
        body,
        grid=(nb,),
        in_specs=[
            pl.BlockSpec((_BN, d), lambda i: (i, 0)),
            pl.BlockSpec((_BN, d), lambda i: (i + nb, 0)),
            pl.BlockSpec((_BN, d), lambda i: (i, 0)),
            pl.BlockSpec((_BN, 1), lambda i: (i, 0)),
            pl.BlockSpec((1, d), lambda i: (0, 0)),
            pl.BlockSpec((d, out), lambda i: (0, 0)),
            pl.BlockSpec((1, out), lambda i: (0, 0)),
            pl.BlockSpec((_BN, out), lambda i: (i, 0)),
        ],
        out_specs=pl.BlockSpec((_BN, out), lambda i: (i, 0)),
        out_shape=jax.ShapeDtypeStruct((n, out), jnp.float32),
    )(s2, s2, g, dinv, b, lw, linb, y)


def kernel(x, edge_index, Ws, bs, linW, linb):
    n, d = x.shape
    e = edge_index.shape[1]
    l_layers, h, _ = Ws.shape
    out_w = linW.shape[1]

    src = edge_index[0].astype(jnp.int32)
    dst = edge_index[1].astype(jnp.int32)

    deg2 = _sc_degree(dst, n, h, 160)
    g, dinv = _tc_first(x, Ws[0], deg2)
    y = jnp.zeros((n, out_w), jnp.float32)

    ch = 160
    for l in range(l_layers):
        s2 = _sc_aggregate(g, src, dst, ch)
        b_l = bs[l].reshape(1, h)
        lw_l = linW[l * h:(l + 1) * h]
        if l < l_layers - 1:
            g, y = _tc_mid(s2, g, dinv, b_l, Ws[l + 1], lw_l, y)
        else:
            return _tc_last(s2, g, dinv, b_l, lw_l, linb.reshape(1, out_w), y)


# speedup vs baseline: 1.1457x; 1.0034x over previous
import functools

import jax
import jax.numpy as jnp
from jax import lax
from jax.experimental import pallas as pl
from jax.experimental.pallas import tpu as pltpu
from jax.experimental.pallas import tpu_sc as plsc

NS = 16
NC = 2


def _sc_degree(dst, n, h, ch):
    e = dst.shape[0]
    round_e = NC * NS * ch
    n_chunks = -(-e // round_e)
    n_chunks += n_chunks % 2
    n_pairs = n_chunks // 2
    epad = round_e * n_chunks
    pad = epad - e
    if pad:
        fill = jnp.arange(pad, dtype=jnp.int32)
        dst = jnp.concatenate([dst, n + (fill % 8)])
    npad = n + 8
    per_tile = epad // (NC * NS)
    rw = 80
    n_rw = n // rw
    mesh = plsc.VectorSubcoreMesh(core_axis_name="c", subcore_axis_name="s")

    @functools.partial(
        pl.kernel,
        out_type=jax.ShapeDtypeStruct((NC * n, h), jnp.float32),
        mesh=mesh,
        scratch_types=[
            pltpu.VMEM((ch,), jnp.int32),
            pltpu.VMEM((ch,), jnp.int32),
            pltpu.VMEM((ch, h), jnp.float32),
            pltpu.VMEM((rw, h), jnp.float32),
            pltpu.VMEM_SHARED((npad, h), jnp.float32),
            pltpu.SemaphoreType.DMA,
            pltpu.SemaphoreType.DMA,
        ],
    )
    def deg_k(dst_hbm, out, didx0, didx1, ones_v, stage, acc,
              sem_i0, sem_i1):
        c = lax.axis_index("c")
        s = lax.axis_index("s")

        def fill_ones(i, carry):
            for j in range(h // 16):
                ones_v[i, pl.ds(j * 16, 16)] = jnp.ones((16,), jnp.float32)
            return carry

        lax.fori_loop(0, ch, fill_ones, 0)

        def fill_zeros(i, carry):
            for j in range(h // 16):
                stage[i, pl.ds(j * 16, 16)] = jnp.zeros((16,), jnp.float32)
            return carry

        lax.fori_loop(0, rw, fill_zeros, 0)
        for kk in range((n_rw + NS - 1) // NS):
            k = s + NS * kk

            @pl.when(k < n_rw)
            def _zero():
                pltpu.sync_copy(stage, acc.at[pl.ds(k * rw, rw)])

        plsc.subcore_barrier()
        base = (c * NS + s) * per_tile

        def eoff(ck):
            return pl.ds(base + ck * ch, ch)

        pltpu.async_copy(dst_hbm.at[eoff(0)], didx0, sem_i0)

        def body(t, carry):
            pltpu.async_copy(dst_hbm.at[eoff(2 * t + 1)], didx1, sem_i1)
            pltpu.make_async_copy(dst_hbm.at[eoff(0)], didx0, sem_i0).wait()
            pltpu.sync_copy(ones_v, acc.at[didx0], add=True)

            @pl.when(t < n_pairs - 1)
            def _prefetch0():
                pltpu.async_copy(dst_hbm.at[eoff(2 * t + 2)], didx0, sem_i0)

            pltpu.make_async_copy(dst_hbm.at[eoff(0)], didx1, sem_i1).wait()
            pltpu.sync_copy(ones_v, acc.at[didx1], add=True)
            return carry

        lax.fori_loop(0, n_pairs, body, 0)
        plsc.subcore_barrier()
        for kk in range((n_rw + NS - 1) // NS):
            k = s + NS * kk

            @pl.when(k < n_rw)
            def _dump():
                pltpu.sync_copy(acc.at[pl.ds(k * rw, rw)], stage)
                off = pl.multiple_of(c * n + k * rw, 8)
                pltpu.sync_copy(stage, out.at[pl.ds(off, rw)])

    return deg_k(dst)


def _sc_aggregate(g, src, dst, ch):
    n, h = g.shape
    e = src.shape[0]
    round_e = NC * NS * ch
    n_chunks = -(-e // round_e)
    n_chunks += n_chunks % 2
    n_pairs = n_chunks // 2
    epad = round_e * n_chunks
    pad = epad - e
    if pad:
        fill = jnp.arange(pad, dtype=jnp.int32)
        src = jnp.concatenate([src, fill % n])
        dst = jnp.concatenate([dst, n + (fill % 8)])
    npad = n + 8
    per_tile = epad // (NC * NS)
    rw = 80
    n_rw = n // rw
    mesh = plsc.VectorSubcoreMesh(core_axis_name="c", subcore_axis_name="s")

    @functools.partial(
        pl.kernel,
        out_type=jax.ShapeDtypeStruct((NC * n, h), jnp.float32),
        mesh=mesh,
        scratch_types=[
            pltpu.VMEM((ch,), jnp.int32),
            pltpu.VMEM((ch,), jnp.int32),
            pltpu.VMEM((ch,), jnp.int32),
            pltpu.VMEM((ch,), jnp.int32),
            pltpu.VMEM((ch, h), jnp.float32),
            pltpu.VMEM((ch, h), jnp.float32),
            pltpu.VMEM_SHARED((npad, h), jnp.float32),
            pltpu.SemaphoreType.DMA,
            pltpu.SemaphoreType.DMA,
            pltpu.SemaphoreType.DMA,
            pltpu.SemaphoreType.DMA,
            pltpu.SemaphoreType.DMA,
            pltpu.SemaphoreType.DMA,
        ],
    )
    def scat_k(g_hbm, src_hbm, dst_hbm, out,
               sidx0, didx0, sidx1, didx1, rows0, rows1, acc,
               sem_g0, sem_g1, sem_i0, sem_i1, sem_s0, sem_s1):
        c = lax.axis_index("c")
        s = lax.axis_index("s")

        def zbody(i, carry):
            for j in range(h // 16):
                rows0[i, pl.ds(j * 16, 16)] = jnp.zeros((16,), jnp.float32)
            return carry

        lax.fori_loop(0, rw, zbody, 0)
        stage = rows0.at[pl.ds(0, rw)]
        for kk in range((n_rw + NS - 1) // NS):
            k = s + NS * kk

            @pl.when(k < n_rw)
            def _zero():
                pltpu.sync_copy(stage, acc.at[pl.ds(k * rw, rw)])

        plsc.subcore_barrier()
        base = (c * NS + s) * per_tile

        def eoff(ck):
            return pl.ds(base + ck * ch, ch)

        def start_idx(ck, sidx, didx, sem):
            pltpu.async_copy(src_hbm.at[eoff(ck)], sidx, sem)
            pltpu.async_copy(dst_hbm.at[eoff(ck)], didx, sem)

        def wait_idx(ck, sidx, didx, sem):
            pltpu.make_async_copy(src_hbm.at[eoff(ck)], sidx, sem).wait()
            pltpu.make_async_copy(dst_hbm.at[eoff(ck)], didx, sem).wait()

        def start_gather(sidx, rows, sem):
            pltpu.async_copy(g_hbm.at[sidx], rows, sem)

        def wait_gather(sidx, rows, sem):
            pltpu.make_async_copy(g_hbm.at[sidx], rows, sem).wait()

        def start_scatter(rows, didx, sem):
            pltpu.async_copy(rows, acc.at[didx], sem, add=True)

        def wait_scatter(rows, didx, sem):
            pltpu.make_async_copy(rows, acc.at[didx], sem).wait()

        start_idx(0, sidx0, didx0, sem_i0)
        wait_idx(0, sidx0, didx0, sem_i0)
        start_gather(sidx0, rows0, sem_g0)

        def body(t, carry):
            @pl.when(t > 0)
            def _drain1():
                wait_scatter(rows1, didx1, sem_s1)

            start_idx(2 * t + 1, sidx1, didx1, sem_i1)
            wait_gather(sidx0, rows0, sem_g0)
            start_scatter(rows0, didx0, sem_s0)
            wait_idx(2 * t + 1, sidx1, didx1, sem_i1)
            start_gather(sidx1, rows1, sem_g1)
            wait_scatter(rows0, didx0, sem_s0)

            @pl.when(t < n_pairs - 1)
            def _prefetch0():
                start_idx(2 * t + 2, sidx0, didx0, sem_i0)

            wait_gather(sidx1, rows1, sem_g1)
            start_scatter(rows1, didx1, sem_s1)

            @pl.when(t < n_pairs - 1)
            def _gather0():
                wait_idx(2 * t + 2, sidx0, didx0, sem_i0)
                start_gather(sidx0, rows0, sem_g0)

            return carry

        lax.fori_loop(0, n_pairs, body, 0)
        wait_scatter(rows1, didx1, sem_s1)
        plsc.subcore_barrier()

        for kk in range((n_rw + NS - 1) // NS):
            k = s + NS * kk

            @pl.when(k < n_rw)
            def _dump():
                pltpu.sync_copy(acc.at[pl.ds(k * rw, rw)], stage)
                off = pl.multiple_of(c * n + k * rw, 8)
                pltpu.sync_copy(stage, out.at[pl.ds(off, rw)])

    return scat_k(g, src, dst)


_BN = 2000


def _tc_matmul(x, w0):
    n, d = x.shape

    def body(x_ref, w_ref, t_ref):
        t_ref[...] = jnp.dot(x_ref[...], w_ref[...],
                             preferred_element_type=jnp.float32)

    return pl.pallas_call(
        body,
        grid=(n // _BN,),
        in_specs=[
            pl.BlockSpec((_BN, d), lambda i: (i, 0)),
            pl.BlockSpec((d, d), lambda i: (0, 0)),
        ],
        out_specs=pl.BlockSpec((_BN, d), lambda i: (i, 0)),
        out_shape=jax.ShapeDtypeStruct((n, d), jnp.float32),
    )(x, w0)


def _tc_first(t, deg2):
    n, d = t.shape
    nb = n // _BN

    def body(t_ref, d0_ref, d1_ref, g_ref, dinv_ref):
        dinv = lax.rsqrt(d0_ref[:, 0:1] + d1_ref[:, 0:1] + 1.0)
        g_ref[...] = dinv * t_ref[...]
        dinv_ref[...] = dinv

    return pl.pallas_call(
        body,
        grid=(nb,),
        in_specs=[
            pl.BlockSpec((_BN, d), lambda i: (i, 0)),
            pl.BlockSpec((_BN, d), lambda i: (i, 0)),
            pl.BlockSpec((_BN, d), lambda i: (i + nb, 0)),
        ],
        out_specs=[
            pl.BlockSpec((_BN, d), lambda i: (i, 0)),
            pl.BlockSpec((_BN, 1), lambda i: (i, 0)),
        ],
        out_shape=[
            jax.ShapeDtypeStruct((n, d), jnp.float32),
            jax.ShapeDtypeStruct((n, 1), jnp.float32),
        ],
    )(t, deg2, deg2)


def _tc_mid(s2, g, dinv, b, w_next, lw, y):
    n, d = g.shape
    nb = n // _BN
    out = lw.shape[1]

    def body(s0_ref, s1_ref, g_ref, dinv_ref, b_ref, w_ref, lw_ref, y_ref,
             gout_ref, yout_ref):
        dinv = dinv_ref[...]
        hh = jnp.maximum(dinv * (s0_ref[...] + s1_ref[...] + g_ref[...])
                         + b_ref[...], 0.0)
        yout_ref[...] = y_ref[...] + jnp.dot(hh, lw_ref[...],
                                             preferred_element_type=jnp.float32)
        gout_ref[...] = dinv * jnp.dot(hh, w_ref[...],
                                       preferred_element_type=jnp.float32)

    return pl.pallas_call(
        body,
        grid=(nb,),
        in_specs=[
            pl.BlockSpec((_BN, d), lambda i: (i, 0)),
            pl.BlockSpec((_BN, d), lambda i: (i + nb, 0)),
            pl.BlockSpec((_BN, d), lambda i: (i, 0)),
            pl.BlockSpec((_BN, 1), lambda i: (i, 0)),
            pl.BlockSpec((1, d), lambda i: (0, 0)),
            pl.BlockSpec((d, d), lambda i: (0, 0)),
            pl.BlockSpec((d, out), lambda i: (0, 0)),
            pl.BlockSpec((_BN, out), lambda i: (i, 0)),
        ],
        out_specs=[
            pl.BlockSpec((_BN, d), lambda i: (i, 0)),
            pl.BlockSpec((_BN, out), lambda i: (i, 0)),
        ],
        out_shape=[
            jax.ShapeDtypeStruct((n, d), jnp.float32),
            jax.ShapeDtypeStruct((n, out), jnp.float32),
        ],
    )(s2, s2, g, dinv, b, w_next, lw, y)


def _tc_last(s2, g, dinv, b, lw, linb, y):
    n, d = g.shape
    nb = n // _BN
    out = lw.shape[1]

    def body(s0_ref, s1_ref, g_ref, dinv_ref, b_ref, lw_ref, lb_ref, y_ref,
             o_ref):
        dinv = dinv_ref[...]
        hh = jnp.maximum(dinv * (s0_ref[...] + s1_ref[...] + g_ref[...])
                         + b_ref[...], 0.0)
        o_ref[...] = (y_ref[...] + lb_ref[...]
                      + jnp.dot(hh, lw_ref[...],
                                preferred_element_type=jnp.float32))

    return pl.pallas_call(
        body,
        grid=(nb,),
        in_specs=[
            pl.BlockSpec((_BN, d), lambda i: (i, 0)),
            pl.BlockSpec((_BN, d), lambda i: (i + nb, 0)),
            pl.BlockSpec((_BN, d), lambda i: (i, 0)),
            pl.BlockSpec((_BN, 1), lambda i: (i, 0)),
            pl.BlockSpec((1, d), lambda i: (0, 0)),
            pl.BlockSpec((d, out), lambda i: (0, 0)),
            pl.BlockSpec((1, out), lambda i: (0, 0)),
            pl.BlockSpec((_BN, out), lambda i: (i, 0)),
        ],
        out_specs=pl.BlockSpec((_BN, out), lambda i: (i, 0)),
        out_shape=jax.ShapeDtypeStruct((n, out), jnp.float32),
    )(s2, s2, g, dinv, b, lw, linb, y)


def kernel(x, edge_index, Ws, bs, linW, linb):
    n, d = x.shape
    e = edge_index.shape[1]
    l_layers, h, _ = Ws.shape
    out_w = linW.shape[1]

    src = edge_index[0].astype(jnp.int32)
    dst = edge_index[1].astype(jnp.int32)

    t0 = _tc_matmul(x, Ws[0])
    deg2 = _sc_degree(dst, n, h, 160)
    g, dinv = _tc_first(t0, deg2)
    y = jnp.zeros((n, out_w), jnp.float32)

    ch = 160
    for l in range(l_layers):
        s2 = _sc_aggregate(g, src, dst, ch)
        b_l = bs[l].reshape(1, h)
        lw_l = linW[l * h:(l + 1) * h]
        if l < l_layers - 1:
            g, y = _tc_mid(s2, g, dinv, b_l, Ws[l + 1], lw_l, y)
        else:
            return _tc_last(s2, g, dinv, b_l, lw_l, linb.reshape(1, out_w), y)
